# baseline (speedup 1.0000x reference)
import functools

import jax
import jax.numpy as jnp
import numpy as np
from jax import lax
from jax.experimental import pallas as pl
from jax.experimental.pallas import tpu as pltpu

N = 10000
E = 160000
EMB = 256
H = 4
HD = 64

_INTERPRET = False


def _mm_bias_kernel(x_ref, w_ref, b_ref, o_ref):
    o_ref[...] = (
        lax.dot_general(x_ref[...], w_ref[...], (((1,), (0,)), ((), ())),
                        preferred_element_type=jnp.float32)
        + b_ref[...]
    )


def _mm_bias(x, w, b, block_rows):
    m, k = x.shape
    n = w.shape[1]
    grid = m // block_rows
    return pl.pallas_call(
        _mm_bias_kernel,
        grid=(grid,),
        in_specs=[
            pl.BlockSpec((block_rows, k), lambda i: (i, 0)),
            pl.BlockSpec((k, n), lambda i: (0, 0)),
            pl.BlockSpec((1, n), lambda i: (0, 0)),
        ],
        out_specs=pl.BlockSpec((block_rows, n), lambda i: (i, 0)),
        out_shape=jax.ShapeDtypeStruct((m, n), jnp.float32),
        interpret=_INTERPRET,
    )(x, w, b.reshape(1, n))


def _edge_mlp_kernel(ea_ref, w1_ref, b1_ref, w2_ref, b2_ref, o_ref):
    mid = (
        lax.dot_general(ea_ref[...], w1_ref[...], (((1,), (0,)), ((), ())),
                        preferred_element_type=jnp.float32)
        + b1_ref[...]
    )
    mid = jnp.maximum(mid, 0.0)
    o_ref[...] = (
        lax.dot_general(mid, w2_ref[...], (((1,), (0,)), ((), ())),
                        preferred_element_type=jnp.float32)
        + b2_ref[...]
    )


def _edge_mlp(edge_attr, w1, b1, w2, b2, block_rows=2000):
    m, k = edge_attr.shape
    n = w2.shape[1]
    grid = m // block_rows
    return pl.pallas_call(
        _edge_mlp_kernel,
        grid=(grid,),
        in_specs=[
            pl.BlockSpec((block_rows, k), lambda i: (i, 0)),
            pl.BlockSpec((k, w1.shape[1]), lambda i: (0, 0)),
            pl.BlockSpec((1, w1.shape[1]), lambda i: (0, 0)),
            pl.BlockSpec((w2.shape[0], n), lambda i: (0, 0)),
            pl.BlockSpec((1, n), lambda i: (0, 0)),
        ],
        out_specs=pl.BlockSpec((block_rows, n), lambda i: (i, 0)),
        out_shape=jax.ShapeDtypeStruct((m, n), jnp.float32),
        interpret=_INTERPRET,
    )(edge_attr, w1, b1.reshape(1, -1), w2, b2.reshape(1, -1))


def _out_proj_kernel(seg_ref, w_ref, b_ref, x_ref, o_ref, *, apply_relu):
    o = (
        lax.dot_general(seg_ref[...], w_ref[...], (((1,), (0,)), ((), ())),
                        preferred_element_type=jnp.float32)
        + b_ref[...]
        + x_ref[...]
    )
    if apply_relu:
        o = jnp.maximum(o, 0.0)
    o_ref[...] = o


def _out_proj(seg, w, b, x, apply_relu, block_rows=2000):
    m, k = seg.shape
    n = w.shape[1]
    grid = m // block_rows
    return pl.pallas_call(
        functools.partial(_out_proj_kernel, apply_relu=apply_relu),
        grid=(grid,),
        in_specs=[
            pl.BlockSpec((block_rows, k), lambda i: (i, 0)),
            pl.BlockSpec((k, n), lambda i: (0, 0)),
            pl.BlockSpec((1, n), lambda i: (0, 0)),
            pl.BlockSpec((block_rows, n), lambda i: (i, 0)),
        ],
        out_specs=pl.BlockSpec((block_rows, n), lambda i: (i, 0)),
        out_shape=jax.ShapeDtypeStruct((m, n), jnp.float32),
        interpret=_INTERPRET,
    )(seg, w, b.reshape(1, n), x)


def _block_diag4(w):
    z = jnp.zeros_like(w)
    rows = []
    for i in range(H):
        rows.append(jnp.concatenate(
            [w if j == i else z for j in range(H)], axis=1))
    return jnp.concatenate(rows, axis=0)


def _fold_weights(W_kqv, b_kqv, W_a1, b_a1):
    w_a1k = W_a1[0:HD]
    w_a1q = W_a1[HD:2 * HD]
    bd_k = _block_diag4(w_a1k) / np.sqrt(HD)
    bd_q = _block_diag4(w_a1q)
    W_q = W_kqv[:, 0:EMB]
    W_k = W_kqv[:, EMB:2 * EMB]
    W_v = W_kqv[:, 2 * EMB:3 * EMB]
    Wbig = jnp.concatenate([W_q @ bd_q, W_k @ bd_k, W_v], axis=1)
    bbig = jnp.concatenate(
        [b_kqv[0:EMB] @ bd_q, b_kqv[EMB:2 * EMB] @ bd_k, b_kqv[2 * EMB:]],
        axis=0)
    return Wbig, bbig


def _edge_softmax_scatter(Aq, Ak, Vv, AeB, src, dst, W_a2, b_a2):
    pre = Ak[src] + Aq[dst] + AeB
    h = jnp.maximum(pre, 0.0).reshape(E, H, HD)
    logit = jnp.einsum("ehd,d->eh", h, W_a2[:, 0]) + b_a2[0]
    m = jax.ops.segment_max(logit, dst, num_segments=N)
    m = jnp.where(jnp.isfinite(m), m, 0.0)
    e = jnp.exp(logit - m[dst])
    s = jax.ops.segment_sum(e, dst, num_segments=N)
    alpha = e / (s[dst] + 1e-16)
    msg = (alpha[:, :, None] * Vv[src].reshape(E, H, HD)).reshape(E, EMB)
    return jax.ops.segment_sum(msg, dst, num_segments=N)


def _layer(x, src, dst, edge_attr, W_kqv, b_kqv, W_edge, b_edge,
           W_a1, b_a1, W_a2, b_a2, W_out, b_out, apply_relu):
    Wbig, bbig = _fold_weights(W_kqv, b_kqv, W_a1, b_a1)
    P = _mm_bias(x, Wbig, bbig, block_rows=400)
    Aq = P[:, 0:EMB]
    Ak = P[:, EMB:2 * EMB]
    Vv = P[:, 2 * EMB:]
    bd_e = _block_diag4(W_a1[2 * HD:3 * HD])
    b_a1_tiled = jnp.tile(b_a1, H)
    AeB = _edge_mlp(edge_attr, W_edge, b_edge, bd_e, b_a1_tiled)
    seg = _edge_softmax_scatter(Aq, Ak, Vv, AeB, src, dst, W_a2, b_a2)
    return _out_proj(seg, W_out, b_out, x, apply_relu)


def kernel(x, edge_index, edge_attr,
           l0_W_kqv, l0_b_kqv, l0_W_edge, l0_b_edge, l0_W_a1, l0_b_a1,
           l0_W_a2, l0_b_a2, l0_W_out, l0_b_out,
           l1_W_kqv, l1_b_kqv, l1_W_edge, l1_b_edge, l1_W_a1, l1_b_a1,
           l1_W_a2, l1_b_a2, l1_W_out, l1_b_out):
    src = edge_index[0]
    dst = edge_index[1]
    x = _layer(x, src, dst, edge_attr, l0_W_kqv, l0_b_kqv, l0_W_edge,
               l0_b_edge, l0_W_a1, l0_b_a1, l0_W_a2, l0_b_a2, l0_W_out,
               l0_b_out, apply_relu=True)
    x = _layer(x, src, dst, edge_attr, l1_W_kqv, l1_b_kqv, l1_W_edge,
               l1_b_edge, l1_W_a1, l1_b_a1, l1_W_a2, l1_b_a2, l1_W_out,
               l1_b_out, apply_relu=False)
    return x
